# Initial kernel scaffold; baseline (speedup 1.0000x reference)
import jax
import jax.numpy as jnp
from jax.experimental import pallas as pl


def kernel(node_emb, rel_emb, src, dst, rel_idx):
    raise NotImplementedError("write your pallas kernel here")


# speedup vs baseline: 3.8283x; 3.8283x over previous
import functools

import jax
import jax.numpy as jnp
from jax import lax
from jax.experimental import pallas as pl
from jax.experimental.pallas import tpu as pltpu
from jax.experimental.pallas import tpu_sc as plsc

_N_EDGES = 320000
_HID = 128
_D2 = _HID // 2
_NW = 32
_EDGES_PER_W = _N_EDGES // _NW
_C = 80
_NCHUNK = _EDGES_PER_W // _C
_GROUPS = _C // 16


def _edge_score_body(node_hbm, rel_hbm, src_hbm, dst_hbm, ridx_hbm, out_hbm,
                     src_v, dst_v, ridx_v, head_v, tail_v, rel_v, out_v, sem):
    cid = lax.axis_index("c")
    sid = lax.axis_index("s")
    wid = sid * 2 + cid
    base = wid * _EDGES_PER_W

    def chunk_body(c, carry):
        be = base + c * _C
        c1 = pltpu.async_copy(src_hbm.at[pl.ds(be, _C)], src_v, sem)
        c2 = pltpu.async_copy(dst_hbm.at[pl.ds(be, _C)], dst_v, sem)
        c3 = pltpu.async_copy(ridx_hbm.at[pl.ds(be, _C)], ridx_v, sem)
        c1.wait()
        c2.wait()
        c3.wait()
        g1 = pltpu.async_copy(node_hbm.at[src_v], head_v, sem)
        g2 = pltpu.async_copy(node_hbm.at[dst_v], tail_v, sem)
        g3 = pltpu.async_copy(rel_hbm.at[ridx_v], rel_v, sem)
        g1.wait()
        g2.wait()
        g3.wait()

        lane = lax.iota(jnp.int32, 16)

        def group_body(g, carry2):
            vec = jnp.zeros((16,), jnp.float32)
            for j in range(16):
                k = g * 16 + j
                acc = jnp.zeros((16,), jnp.float32)
                for q in range(_D2 // 16):
                    lo = q * 16
                    hi = _D2 + q * 16
                    h_i = head_v[k, pl.ds(lo, 16)]
                    h_j = head_v[k, pl.ds(hi, 16)]
                    t_i = tail_v[k, pl.ds(lo, 16)]
                    t_j = tail_v[k, pl.ds(hi, 16)]
                    r_f = rel_v[k, pl.ds(lo, 16)]
                    r_b = rel_v[k, pl.ds(hi, 16)]
                    acc = acc + h_i * r_f * t_j + t_i * r_b * h_j
                s = jnp.full((16,), jnp.sum(acc))
                vec = jnp.where(lane == j, s, vec)
            out_v[pl.ds(g * 16, 16)] = jnp.clip(0.5 * vec, -20.0, 20.0)
            return carry2

        lax.fori_loop(0, _GROUPS, group_body, 0)
        pltpu.sync_copy(out_v, out_hbm.at[pl.ds(be, _C)])
        return carry

    lax.fori_loop(0, _NCHUNK, chunk_body, 0)


@jax.jit
def _sc_edge_score(node_emb, rel_emb, src, dst, rel_idx):
    mesh = plsc.VectorSubcoreMesh(core_axis_name="c", subcore_axis_name="s")
    run = pl.kernel(
        _edge_score_body,
        mesh=mesh,
        compiler_params=pltpu.CompilerParams(needs_layout_passes=False),
        out_type=jax.ShapeDtypeStruct((_N_EDGES,), jnp.float32),
        scratch_types=[
            pltpu.VMEM((_C,), jnp.int32),
            pltpu.VMEM((_C,), jnp.int32),
            pltpu.VMEM((_C,), jnp.int32),
            pltpu.VMEM((_C, _HID), jnp.float32),
            pltpu.VMEM((_C, _HID), jnp.float32),
            pltpu.VMEM((_C, _HID), jnp.float32),
            pltpu.VMEM((_C,), jnp.float32),
            pltpu.SemaphoreType.DMA,
        ],
    )
    return run(node_emb, rel_emb, src, dst, rel_idx)


def kernel(node_emb, rel_emb, src, dst, rel_idx):
    return _sc_edge_score(node_emb, rel_emb,
                          src.astype(jnp.int32), dst.astype(jnp.int32),
                          rel_idx.astype(jnp.int32))
